# baseline (speedup 1.0000x reference)
import functools

import jax
import jax.numpy as jnp
from jax.experimental import pallas as pl

H, DH, E, TOPK = 16, 64, 16, 8
EPS = 1e-6
THETA = 10000.0
NEG = -1e30


def _qkv_gate_body(x_ref, wg_ref, bg_ref, wq_ref, wk_ref, wv_ref, cos_ref,
                   sin_ref, gate_ref, q_ref, k_ref, v_ref):
    x = x_ref[...]
    bq = x.shape[0]
    logits = jnp.dot(x, wg_ref[...], preferred_element_type=jnp.float32)
    logits = logits + bg_ref[...]
    mx = jnp.max(logits, axis=1, keepdims=True)
    p = jnp.exp(logits - mx)
    sm = p / jnp.sum(p, axis=1, keepdims=True)
    iota = jax.lax.broadcasted_iota(jnp.int32, (bq, E), 1)
    cur = sm
    mask = jnp.zeros((bq, E), dtype=jnp.float32)
    for _ in range(TOPK):
        m = jnp.max(cur, axis=1, keepdims=True)
        cand = cur == m
        first = jnp.min(jnp.where(cand, iota, E), axis=1, keepdims=True)
        sel = iota == first
        mask = jnp.where(sel, 1.0, mask)
        cur = jnp.where(sel, -1.0, cur)
    masked = sm * mask
    gate_ref[...] = masked / (masked + EPS)
    cos = cos_ref[...]
    sin = sin_ref[...]
    xq = jnp.dot(x, wq_ref[...], preferred_element_type=jnp.float32)
    xk = jnp.dot(x, wk_ref[...], preferred_element_type=jnp.float32)
    xv = jnp.dot(x, wv_ref[...], preferred_element_type=jnp.float32)
    half = DH // 2
    for h in range(H):
        b = h * DH
        q1 = xq[:, b:b + half]
        q2 = xq[:, b + half:b + DH]
        q_ref[h, :, :half] = q1 * cos - q2 * sin
        q_ref[h, :, half:] = q2 * cos + q1 * sin
        k1 = xk[:, b:b + half]
        k2 = xk[:, b + half:b + DH]
        k_ref[h, :, :half] = k1 * cos - k2 * sin
        k_ref[h, :, half:] = k2 * cos + k1 * sin
        v_ref[h, :, :] = xv[:, b:b + DH]


def _attn_body(q_ref, k_ref, v_ref, g_ref, o_ref, *, bq, bk):
    h = pl.program_id(0)
    qi = pl.program_id(1)
    scale = 1.0 / (DH ** 0.5)
    q = q_ref[0] * scale

    def body(j, carry):
        acc, m, l = carry
        kj = k_ref[0, pl.ds(j * bk, bk), :]
        vj = v_ref[0, pl.ds(j * bk, bk), :]
        s = jax.lax.dot_general(q, kj, (((1,), (1,)), ((), ())),
                                preferred_element_type=jnp.float32)
        rows = qi * bq + jax.lax.broadcasted_iota(jnp.int32, (bq, bk), 0)
        cols = j * bk + jax.lax.broadcasted_iota(jnp.int32, (bq, bk), 1)
        s = jnp.where(rows >= cols, s, NEG)
        m2 = jnp.maximum(m, jnp.max(s, axis=1, keepdims=True))
        alpha = jnp.exp(m - m2)
        pexp = jnp.exp(s - m2)
        l2 = l * alpha + jnp.sum(pexp, axis=1, keepdims=True)
        acc2 = acc * alpha + jnp.dot(pexp, vj, preferred_element_type=jnp.float32)
        return acc2, m2, l2

    nsteps = (qi * bq) // bk + 1
    acc0 = jnp.zeros((bq, DH), dtype=jnp.float32)
    m0 = jnp.full((bq, 1), NEG, dtype=jnp.float32)
    l0 = jnp.zeros((bq, 1), dtype=jnp.float32)
    acc, _, l = jax.lax.fori_loop(0, nsteps, body, (acc0, m0, l0))
    ctx = acc / l
    hiota = jax.lax.broadcasted_iota(jnp.int32, (bq, E), 1)
    g = jnp.sum(jnp.where(hiota == h, g_ref[...], 0.0), axis=1, keepdims=True)
    o_ref[0] = ctx * g


def _outproj_body(ctx_ref, wo_ref, o_ref):
    parts = [ctx_ref[h] for h in range(H)]
    cat = jnp.concatenate(parts, axis=1)
    o_ref[...] = jnp.dot(cat, wo_ref[...], preferred_element_type=jnp.float32)


def kernel(X, Wg, bg, Wq, Wk, Wv, Wo):
    b, s, d = X.shape
    x = X.reshape(s, d)
    bq = 256
    bk = 256
    nq = s // bq
    half = DH // 2
    inv_freq = 1.0 / (THETA ** (jnp.arange(0, DH, 2, dtype=jnp.float32) / DH))
    t = jnp.arange(s, dtype=jnp.float32)
    freqs = jnp.outer(t, inv_freq)
    cos32 = jnp.cos(freqs)
    sin32 = jnp.sin(freqs)
    bg2 = bg.reshape(1, E)

    gate, q, k, v = pl.pallas_call(
        _qkv_gate_body,
        grid=(nq,),
        in_specs=[
            pl.BlockSpec((bq, d), lambda i: (i, 0)),
            pl.BlockSpec((d, E), lambda i: (0, 0)),
            pl.BlockSpec((1, E), lambda i: (0, 0)),
            pl.BlockSpec((d, H * DH), lambda i: (0, 0)),
            pl.BlockSpec((d, H * DH), lambda i: (0, 0)),
            pl.BlockSpec((d, H * DH), lambda i: (0, 0)),
            pl.BlockSpec((bq, half), lambda i: (i, 0)),
            pl.BlockSpec((bq, half), lambda i: (i, 0)),
        ],
        out_specs=[
            pl.BlockSpec((bq, E), lambda i: (i, 0)),
            pl.BlockSpec((H, bq, DH), lambda i: (0, i, 0)),
            pl.BlockSpec((H, bq, DH), lambda i: (0, i, 0)),
            pl.BlockSpec((H, bq, DH), lambda i: (0, i, 0)),
        ],
        out_shape=[
            jax.ShapeDtypeStruct((s, E), jnp.float32),
            jax.ShapeDtypeStruct((H, s, DH), jnp.float32),
            jax.ShapeDtypeStruct((H, s, DH), jnp.float32),
            jax.ShapeDtypeStruct((H, s, DH), jnp.float32),
        ],
    )(x, Wg, bg2, Wq, Wk, Wv, cos32, sin32)

    ctx = pl.pallas_call(
        functools.partial(_attn_body, bq=bq, bk=bk),
        grid=(H, nq),
        in_specs=[
            pl.BlockSpec((1, bq, DH), lambda h, i: (h, i, 0)),
            pl.BlockSpec((1, s, DH), lambda h, i: (h, 0, 0)),
            pl.BlockSpec((1, s, DH), lambda h, i: (h, 0, 0)),
            pl.BlockSpec((bq, E), lambda h, i: (i, 0)),
        ],
        out_specs=pl.BlockSpec((1, bq, DH), lambda h, i: (h, i, 0)),
        out_shape=jax.ShapeDtypeStruct((H, s, DH), jnp.float32),
    )(q, k, v, gate)

    out = pl.pallas_call(
        _outproj_body,
        grid=(nq,),
        in_specs=[
            pl.BlockSpec((H, bq, DH), lambda i: (0, i, 0)),
            pl.BlockSpec((H * DH, d), lambda i: (0, 0)),
        ],
        out_specs=pl.BlockSpec((bq, d), lambda i: (i, 0)),
        out_shape=jax.ShapeDtypeStruct((s, d), jnp.float32),
    )(ctx, Wo)

    return out.reshape(b, s, d)


# speedup vs baseline: 1.9388x; 1.9388x over previous
import functools

import jax
import jax.numpy as jnp
from jax.experimental import pallas as pl

H, DH, E, TOPK = 16, 64, 16, 8
EPS = 1e-6
THETA = 10000.0
NEG = -1e30


def _qkv_gate_body(x_ref, wg_ref, bg_ref, wq_ref, wk_ref, wv_ref, cosq_ref,
                   sinq_ref, cos_ref, sin_ref, gate_ref, q_ref, k_ref, v_ref):
    x = x_ref[...]
    bq = x.shape[0]
    logits = jnp.dot(x, wg_ref[...], preferred_element_type=jnp.float32)
    logits = logits + bg_ref[...]
    mx = jnp.max(logits, axis=1, keepdims=True)
    p = jnp.exp(logits - mx)
    sm = p / jnp.sum(p, axis=1, keepdims=True)
    iota = jax.lax.broadcasted_iota(jnp.int32, (bq, E), 1)
    cur = sm
    mask = jnp.zeros((bq, E), dtype=jnp.float32)
    for _ in range(TOPK):
        m = jnp.max(cur, axis=1, keepdims=True)
        cand = cur == m
        first = jnp.min(jnp.where(cand, iota, E), axis=1, keepdims=True)
        sel = iota == first
        mask = jnp.where(sel, 1.0, mask)
        cur = jnp.where(sel, -1.0, cur)
    masked = sm * mask
    gate_ref[...] = masked / (masked + EPS)
    xb = x.astype(jnp.bfloat16)
    cosq = cosq_ref[...]
    sinq = sinq_ref[...]
    cos = cos_ref[...]
    sin = sin_ref[...]
    xq = jnp.dot(xb, wq_ref[...], preferred_element_type=jnp.float32)
    xk = jnp.dot(xb, wk_ref[...], preferred_element_type=jnp.float32)
    xv = jnp.dot(xb, wv_ref[...], preferred_element_type=jnp.float32)
    half = DH // 2
    for h in range(H):
        b = h * DH
        q1 = xq[:, b:b + half]
        q2 = xq[:, b + half:b + DH]
        q_ref[h, :, :half] = (q1 * cosq - q2 * sinq).astype(jnp.bfloat16)
        q_ref[h, :, half:] = (q2 * cosq + q1 * sinq).astype(jnp.bfloat16)
        k1 = xk[:, b:b + half]
        k2 = xk[:, b + half:b + DH]
        k_ref[h, :, :half] = (k1 * cos - k2 * sin).astype(jnp.bfloat16)
        k_ref[h, :, half:] = (k2 * cos + k1 * sin).astype(jnp.bfloat16)
        v_ref[h, :, :] = xv[:, b:b + DH].astype(jnp.bfloat16)


def _attn_body(q_ref, k_ref, v_ref, g_ref, o_ref, *, bq, bk):
    h = pl.program_id(0)
    qi = pl.program_id(1)
    q = q_ref[0]

    def body(j, carry):
        acc, m, l = carry
        kj = k_ref[0, pl.ds(j * bk, bk), :]
        vj = v_ref[0, pl.ds(j * bk, bk), :]
        s = jax.lax.dot_general(q, kj, (((1,), (1,)), ((), ())),
                                preferred_element_type=jnp.float32)
        m2 = jnp.maximum(m, jnp.max(s, axis=1, keepdims=True))
        alpha = jnp.exp(m - m2)
        pexp = jnp.exp(s - m2)
        l2 = l * alpha + jnp.sum(pexp, axis=1, keepdims=True)
        acc2 = acc * alpha + jnp.dot(pexp.astype(jnp.bfloat16), vj,
                                     preferred_element_type=jnp.float32)
        return acc2, m2, l2

    acc0 = jnp.zeros((bq, DH), dtype=jnp.float32)
    m0 = jnp.full((bq, 1), NEG, dtype=jnp.float32)
    l0 = jnp.zeros((bq, 1), dtype=jnp.float32)
    acc, m, l = jax.lax.fori_loop(0, qi, body, (acc0, m0, l0))
    kq = k_ref[0, pl.ds(qi * bk, bk), :]
    vq = v_ref[0, pl.ds(qi * bk, bk), :]
    s = jax.lax.dot_general(q, kq, (((1,), (1,)), ((), ())),
                            preferred_element_type=jnp.float32)
    rows = jax.lax.broadcasted_iota(jnp.int32, (bq, bk), 0)
    cols = jax.lax.broadcasted_iota(jnp.int32, (bq, bk), 1)
    s = jnp.where(rows >= cols, s, NEG)
    m2 = jnp.maximum(m, jnp.max(s, axis=1, keepdims=True))
    alpha = jnp.exp(m - m2)
    pexp = jnp.exp(s - m2)
    l = l * alpha + jnp.sum(pexp, axis=1, keepdims=True)
    acc = acc * alpha + jnp.dot(pexp.astype(jnp.bfloat16), vq,
                                preferred_element_type=jnp.float32)
    ctx = acc / l
    hiota = jax.lax.broadcasted_iota(jnp.int32, (bq, E), 1)
    g = jnp.sum(jnp.where(hiota == h, g_ref[...], 0.0), axis=1, keepdims=True)
    o_ref[0] = (ctx * g).astype(jnp.bfloat16)


def _outproj_body(ctx_ref, wo_ref, o_ref):
    parts = [ctx_ref[h] for h in range(H)]
    cat = jnp.concatenate(parts, axis=1)
    o_ref[...] = jnp.dot(cat, wo_ref[...], preferred_element_type=jnp.float32)


def kernel(X, Wg, bg, Wq, Wk, Wv, Wo):
    b, s, d = X.shape
    x = X.reshape(s, d)
    bp = 512
    bq = 512
    bk = 512
    np_ = s // bp
    nq = s // bq
    half = DH // 2
    inv_freq = 1.0 / (THETA ** (jnp.arange(0, DH, 2, dtype=jnp.float32) / DH))
    t = jnp.arange(s, dtype=jnp.float32)
    freqs = jnp.outer(t, inv_freq)
    cos32 = jnp.cos(freqs)
    sin32 = jnp.sin(freqs)
    scale = 1.0 / (DH ** 0.5)
    cosq = cos32 * scale
    sinq = sin32 * scale
    bg2 = bg.reshape(1, E)
    wq_b = Wq.astype(jnp.bfloat16)
    wk_b = Wk.astype(jnp.bfloat16)
    wv_b = Wv.astype(jnp.bfloat16)
    wo_b = Wo.astype(jnp.bfloat16)

    gate, q, k, v = pl.pallas_call(
        _qkv_gate_body,
        grid=(np_,),
        in_specs=[
            pl.BlockSpec((bp, d), lambda i: (i, 0)),
            pl.BlockSpec((d, E), lambda i: (0, 0)),
            pl.BlockSpec((1, E), lambda i: (0, 0)),
            pl.BlockSpec((d, H * DH), lambda i: (0, 0)),
            pl.BlockSpec((d, H * DH), lambda i: (0, 0)),
            pl.BlockSpec((d, H * DH), lambda i: (0, 0)),
            pl.BlockSpec((bp, half), lambda i: (i, 0)),
            pl.BlockSpec((bp, half), lambda i: (i, 0)),
            pl.BlockSpec((bp, half), lambda i: (i, 0)),
            pl.BlockSpec((bp, half), lambda i: (i, 0)),
        ],
        out_specs=[
            pl.BlockSpec((bp, E), lambda i: (i, 0)),
            pl.BlockSpec((H, bp, DH), lambda i: (0, i, 0)),
            pl.BlockSpec((H, bp, DH), lambda i: (0, i, 0)),
            pl.BlockSpec((H, bp, DH), lambda i: (0, i, 0)),
        ],
        out_shape=[
            jax.ShapeDtypeStruct((s, E), jnp.float32),
            jax.ShapeDtypeStruct((H, s, DH), jnp.bfloat16),
            jax.ShapeDtypeStruct((H, s, DH), jnp.bfloat16),
            jax.ShapeDtypeStruct((H, s, DH), jnp.bfloat16),
        ],
    )(x, Wg, bg2, wq_b, wk_b, wv_b, cosq, sinq, cos32, sin32)

    ctx = pl.pallas_call(
        functools.partial(_attn_body, bq=bq, bk=bk),
        grid=(H, nq),
        in_specs=[
            pl.BlockSpec((1, bq, DH), lambda h, i: (h, i, 0)),
            pl.BlockSpec((1, s, DH), lambda h, i: (h, 0, 0)),
            pl.BlockSpec((1, s, DH), lambda h, i: (h, 0, 0)),
            pl.BlockSpec((bq, E), lambda h, i: (i, 0)),
        ],
        out_specs=pl.BlockSpec((1, bq, DH), lambda h, i: (h, i, 0)),
        out_shape=jax.ShapeDtypeStruct((H, s, DH), jnp.bfloat16),
    )(q, k, v, gate)

    out = pl.pallas_call(
        _outproj_body,
        grid=(np_,),
        in_specs=[
            pl.BlockSpec((H, bp, DH), lambda i: (0, i, 0)),
            pl.BlockSpec((H * DH, d), lambda i: (0, 0)),
        ],
        out_specs=pl.BlockSpec((bp, d), lambda i: (i, 0)),
        out_shape=jax.ShapeDtypeStruct((s, d), jnp.float32),
    )(ctx, wo_b)

    return out.reshape(b, s, d)
